# baseline (speedup 1.0000x reference)
import functools

import jax
import jax.numpy as jnp
from jax import lax
from jax.experimental import pallas as pl
from jax.experimental.pallas import tpu as pltpu
from jax.experimental.pallas import tpu_sc as plsc

N_ATOMS = 10000
HIDDEN = 256
NUM_RADIAL = 6
CUTOFF = 8.0
MAX_NEIGH = 32
E_TOTAL = N_ATOMS * MAX_NEIGH

SC_WORKERS = 32
SC_CHUNK = 128
SC_CHUNKS_PER_W = 80
E_PAD = SC_WORKERS * SC_CHUNKS_PER_W * SC_CHUNK

R_EDGE = 80
R_MM = 400

_ENV_A = -28.0
_ENV_B = 48.0
_ENV_C = -21.0


def _sc_gather(table, idx_pad):
    mesh = plsc.VectorSubcoreMesh(core_axis_name="c", subcore_axis_name="s")

    @functools.partial(
        pl.kernel,
        mesh=mesh,
        out_type=jax.ShapeDtypeStruct((E_PAD, HIDDEN), jnp.float32),
        scratch_types=[
            pltpu.VMEM((SC_CHUNK,), jnp.int32),
            pltpu.VMEM((SC_CHUNK, HIDDEN), jnp.float32),
            pltpu.SemaphoreType.DMA,
        ],
    )
    def gather_kernel(table_hbm, idx_hbm, out_hbm, idx_v, rows_v, sem):
        wid = lax.axis_index("s") * 2 + lax.axis_index("c")

        def body(t, carry):
            base = (wid * SC_CHUNKS_PER_W + t) * SC_CHUNK
            pltpu.sync_copy(idx_hbm.at[pl.ds(base, SC_CHUNK)], idx_v)
            pltpu.async_copy(table_hbm.at[idx_v], rows_v, sem).wait()
            pltpu.sync_copy(rows_v, out_hbm.at[pl.ds(base, SC_CHUNK)])
            return carry

        lax.fori_loop(0, SC_CHUNKS_PER_W, body, 0)

    return gather_kernel(table, idx_pad)


def _proj_kernel(x, w):
    n, k = x.shape
    m = w.shape[1]

    def body(x_ref, w_ref, o_ref):
        o_ref[...] = jnp.dot(x_ref[...], w_ref[...],
                             preferred_element_type=jnp.float32)

    return pl.pallas_call(
        body,
        grid=(n // R_MM,),
        in_specs=[
            pl.BlockSpec((R_MM, k), lambda g: (g, 0)),
            pl.BlockSpec((k, m), lambda g: (0, 0)),
        ],
        out_specs=pl.BlockSpec((R_MM, m), lambda g: (g, 0)),
        out_shape=jax.ShapeDtypeStruct((n, m), jnp.float32),
    )(x, w)


def _edge_combine(a, bg_pad, dist_col, valid_col, w1c, b1row, freq):
    re = R_EDGE * MAX_NEIGH

    def body(a_ref, bg_ref, d_ref, v_ref, w1c_ref, b1_ref, f_ref,
             hs_ref, cnt_ref):
        x = d_ref[...] * (1.0 / CUTOFF)
        inv = 1.0 / x
        x2 = x * x
        x5 = x2 * x2 * x
        env = (inv + _ENV_A * x5 + _ENV_B * x5 * x + _ENV_C * x5 * x2)
        env = jnp.where(x < 1.0, env, 0.0)
        arep = jnp.broadcast_to(
            a_ref[...][:, None, :], (R_EDGE, MAX_NEIGH, HIDDEN)
        ).reshape(re, HIDDEN)
        acc = bg_ref[...] + b1_ref[...] + arep
        for r in range(NUM_RADIAL):
            s = env * jnp.sin(f_ref[0, r] * x)
            acc = acc + s * w1c_ref[r:r + 1, :]
        h = acc * jax.nn.sigmoid(acc)
        h = h * v_ref[...]
        hs_ref[...] = h.reshape(R_EDGE, MAX_NEIGH, HIDDEN).sum(axis=1)
        cnt_ref[...] = v_ref[...].reshape(R_EDGE, MAX_NEIGH).sum(
            axis=1, keepdims=True)

    return pl.pallas_call(
        body,
        grid=(N_ATOMS // R_EDGE,),
        in_specs=[
            pl.BlockSpec((R_EDGE, HIDDEN), lambda g: (g, 0)),
            pl.BlockSpec((re, HIDDEN), lambda g: (g, 0)),
            pl.BlockSpec((re, 1), lambda g: (g, 0)),
            pl.BlockSpec((re, 1), lambda g: (g, 0)),
            pl.BlockSpec((8, HIDDEN), lambda g: (0, 0)),
            pl.BlockSpec((1, HIDDEN), lambda g: (0, 0)),
            pl.BlockSpec((1, 8), lambda g: (0, 0),
                         memory_space=pltpu.SMEM),
        ],
        out_specs=[
            pl.BlockSpec((R_EDGE, HIDDEN), lambda g: (g, 0)),
            pl.BlockSpec((R_EDGE, 1), lambda g: (g, 0)),
        ],
        out_shape=[
            jax.ShapeDtypeStruct((N_ATOMS, HIDDEN), jnp.float32),
            jax.ShapeDtypeStruct((N_ATOMS, 1), jnp.float32),
        ],
    )(a, bg_pad, dist_col, valid_col, w1c, b1row, freq)


def _update(x, hs, cnt, w2t, b2row, u1at, u1bt, ub1row, u2t, ub2row,
            wab_next=None):
    with_next = wab_next is not None

    def body(x_ref, hs_ref, cnt_ref, w2_ref, b2_ref, u1a_ref, u1b_ref,
             ub1_ref, u2_ref, ub2_ref, *rest):
        if with_next:
            wab_ref, o_ref, ab_ref = rest
        else:
            (o_ref,) = rest
        xb = x_ref[...]
        agg = jnp.dot(hs_ref[...], w2_ref[...],
                      preferred_element_type=jnp.float32)
        agg = agg + cnt_ref[...] * b2_ref[...]
        u = (jnp.dot(xb, u1a_ref[...], preferred_element_type=jnp.float32)
             + jnp.dot(agg, u1b_ref[...], preferred_element_type=jnp.float32)
             + ub1_ref[...])
        u = u * jax.nn.sigmoid(u)
        xn = jnp.dot(u, u2_ref[...],
                     preferred_element_type=jnp.float32) + ub2_ref[...] + xb
        if with_next:
            o_ref[...] = xn
            ab_ref[...] = jnp.dot(xn, wab_ref[...],
                                  preferred_element_type=jnp.float32)
        else:
            o_ref[...] = xn

    h = HIDDEN
    in_specs = [
        pl.BlockSpec((R_MM, h), lambda g: (g, 0)),
        pl.BlockSpec((R_MM, h), lambda g: (g, 0)),
        pl.BlockSpec((R_MM, 1), lambda g: (g, 0)),
        pl.BlockSpec((h, h), lambda g: (0, 0)),
        pl.BlockSpec((1, h), lambda g: (0, 0)),
        pl.BlockSpec((h, h), lambda g: (0, 0)),
        pl.BlockSpec((h, h), lambda g: (0, 0)),
        pl.BlockSpec((1, h), lambda g: (0, 0)),
        pl.BlockSpec((h, h), lambda g: (0, 0)),
        pl.BlockSpec((1, h), lambda g: (0, 0)),
    ]
    inputs = [x, hs, cnt, w2t, b2row, u1at, u1bt, ub1row, u2t, ub2row]
    if with_next:
        in_specs.append(pl.BlockSpec((h, 2 * h), lambda g: (0, 0)))
        inputs.append(wab_next)
        out_specs = [pl.BlockSpec((R_MM, h), lambda g: (g, 0)),
                     pl.BlockSpec((R_MM, 2 * h), lambda g: (g, 0))]
        out_shape = [jax.ShapeDtypeStruct((N_ATOMS, h), jnp.float32),
                     jax.ShapeDtypeStruct((N_ATOMS, 2 * h), jnp.float32)]
    else:
        out_specs = pl.BlockSpec((R_MM, h), lambda g: (g, 0))
        out_shape = jax.ShapeDtypeStruct((N_ATOMS, h), jnp.float32)

    return pl.pallas_call(
        body,
        grid=(N_ATOMS // R_MM,),
        in_specs=in_specs,
        out_specs=out_specs,
        out_shape=out_shape,
    )(*inputs)


def _radius_graph(pos, batch):
    n = pos.shape[0]
    sq = jnp.sum(pos * pos, axis=1)
    all_idx = jnp.arange(n)
    cols, valids = [], []
    chunk = 2000
    for start in range(0, n, chunk):
        end = start + chunk
        pc = pos[start:end]
        d2 = sq[start:end, None] + sq[None, :] - 2.0 * (pc @ pos.T)
        d2 = jnp.maximum(d2, 0.0)
        same = batch[start:end, None] == batch[None, :]
        not_self = all_idx[start:end, None] != all_idx[None, :]
        mask = same & not_self & (d2 < CUTOFF * CUTOFF)
        d2m = jnp.where(mask, d2, jnp.inf)
        vals, idxs = jax.lax.top_k(-d2m, MAX_NEIGH)
        cols.append(idxs)
        valids.append(vals > -jnp.inf)
    return jnp.concatenate(cols), jnp.concatenate(valids)


def kernel(atom_features, coordinates, batch_id, rbf_freq,
           i0_W1, i0_b1, i0_W2, i0_b2, i1_W1, i1_b1, i1_W2, i1_b2,
           u0_W1, u0_b1, u0_W2, u0_b2, u1_W1, u1_b1, u1_W2, u1_b2):
    h = HIDDEN
    j_idx, valid = _radius_graph(coordinates, batch_id)
    jf = j_idx.reshape(-1)
    diff = coordinates[:, None, :] - coordinates[jf].reshape(
        N_ATOMS, MAX_NEIGH, 3)
    dist = jnp.sqrt(jnp.sum(diff * diff, axis=-1))
    dist = jnp.where(valid, dist, 0.5 * CUTOFF)
    dist_col = dist.reshape(E_TOTAL, 1)
    valid_col = valid.astype(jnp.float32).reshape(E_TOTAL, 1)
    j_pad = jnp.concatenate(
        [jf, jnp.zeros((E_PAD - E_TOTAL,), dtype=jf.dtype)])
    freq = jnp.zeros((1, 8), jnp.float32).at[0, :NUM_RADIAL].set(rbf_freq)

    inter = [(i0_W1, i0_b1, i0_W2, i0_b2), (i1_W1, i1_b1, i1_W2, i1_b2)]
    upd = [(u0_W1, u0_b1, u0_W2, u0_b2), (u1_W1, u1_b1, u1_W2, u1_b2)]

    def wab_of(W1):
        return jnp.concatenate([W1[:, :h].T, W1[:, h:2 * h].T], axis=1)

    x = atom_features
    ab = _proj_kernel(x, wab_of(inter[0][0]))
    for blk in range(2):
        W1, b1, W2, b2 = inter[blk]
        uW1, ub1, uW2, ub2 = upd[blk]
        w1c = jnp.zeros((8, h), jnp.float32).at[:NUM_RADIAL, :].set(
            W1[:, 2 * h:2 * h + NUM_RADIAL].T)
        a = ab[:, :h]
        b = ab[:, h:]
        bg_pad = _sc_gather(b, j_pad)
        hs, cnt = _edge_combine(a, bg_pad, dist_col, valid_col,
                                w1c, b1[None, :], freq)
        wab_next = wab_of(inter[1][0]) if blk == 0 else None
        res = _update(x, hs, cnt, W2.T, b2[None, :], uW1[:, :h].T,
                      uW1[:, h:].T, ub1[None, :], uW2.T, ub2[None, :],
                      wab_next)
        if blk == 0:
            x, ab = res
        else:
            x = res

    n_valid = jnp.sum(valid)
    return jnp.where(n_valid > 0, x, atom_features)


# speedup vs baseline: 1.6962x; 1.6962x over previous
import functools

import jax
import jax.numpy as jnp
from jax import lax
from jax.experimental import pallas as pl
from jax.experimental.pallas import tpu as pltpu
from jax.experimental.pallas import tpu_sc as plsc

N_ATOMS = 10000
HIDDEN = 256
NUM_RADIAL = 6
CUTOFF = 8.0
MAX_NEIGH = 32
E_TOTAL = N_ATOMS * MAX_NEIGH

SC_WORKERS = 32
SC_CHUNK = 128
SC_CHUNKS_PER_W = 80
E_PAD = SC_WORKERS * SC_CHUNKS_PER_W * SC_CHUNK

R_EDGE = 80
R_MM = 400

R_GRAPH = 200
W_SEG = 384
C_WIN = 1024

_ENV_A = -28.0
_ENV_B = 48.0
_ENV_C = -21.0


def _sc_gather(table, idx_pad):
    mesh = plsc.VectorSubcoreMesh(core_axis_name="c", subcore_axis_name="s")

    @functools.partial(
        pl.kernel,
        mesh=mesh,
        out_type=jax.ShapeDtypeStruct((E_PAD, HIDDEN), jnp.float32),
        scratch_types=[
            pltpu.VMEM((SC_CHUNK,), jnp.int32),
            pltpu.VMEM((SC_CHUNK, HIDDEN), jnp.float32),
            pltpu.SemaphoreType.DMA,
        ],
    )
    def gather_kernel(table_hbm, idx_hbm, out_hbm, idx_v, rows_v, sem):
        wid = lax.axis_index("s") * 2 + lax.axis_index("c")

        def body(t, carry):
            base = (wid * SC_CHUNKS_PER_W + t) * SC_CHUNK
            pltpu.sync_copy(idx_hbm.at[pl.ds(base, SC_CHUNK)], idx_v)
            pltpu.async_copy(table_hbm.at[idx_v], rows_v, sem).wait()
            pltpu.sync_copy(rows_v, out_hbm.at[pl.ds(base, SC_CHUNK)])
            return carry

        lax.fori_loop(0, SC_CHUNKS_PER_W, body, 0)

    return gather_kernel(table, idx_pad)


def _proj_kernel(x, w):
    n, k = x.shape
    m = w.shape[1]

    def body(x_ref, w_ref, o_ref):
        o_ref[...] = jnp.dot(x_ref[...], w_ref[...],
                             preferred_element_type=jnp.float32)

    return pl.pallas_call(
        body,
        grid=(n // R_MM,),
        in_specs=[
            pl.BlockSpec((R_MM, k), lambda g: (g, 0)),
            pl.BlockSpec((k, m), lambda g: (0, 0)),
        ],
        out_specs=pl.BlockSpec((R_MM, m), lambda g: (g, 0)),
        out_shape=jax.ShapeDtypeStruct((n, m), jnp.float32),
    )(x, w)


def _edge_combine(a, bg_pad, dist_row, valid_col, w1c, b1row, freq):
    re = R_EDGE * MAX_NEIGH

    def body(a_ref, bg_ref, d_ref, v_ref, w1c_ref, b1_ref, f_ref,
             hs_ref, cnt_ref):
        x = d_ref[...] * (1.0 / CUTOFF)
        inv = 1.0 / x
        x2 = x * x
        x5 = x2 * x2 * x
        env = (inv + _ENV_A * x5 + _ENV_B * x5 * x + _ENV_C * x5 * x2)
        env = jnp.where(x < 1.0, env, 0.0)
        rows = [env * jnp.sin(f_ref[0, r] * x) for r in range(NUM_RADIAL)]
        rows.append(jnp.zeros((2, re), jnp.float32))
        rbf8 = jnp.concatenate(rows, axis=0)
        rbfp = jax.lax.dot_general(
            rbf8, w1c_ref[...], (((0,), (0,)), ((), ())),
            preferred_element_type=jnp.float32)
        arep = jnp.broadcast_to(
            a_ref[...][:, None, :], (R_EDGE, MAX_NEIGH, HIDDEN)
        ).reshape(re, HIDDEN)
        acc = bg_ref[...] + b1_ref[...] + arep + rbfp
        h = acc * jax.nn.sigmoid(acc)
        h = h * v_ref[...]
        hs_ref[...] = h.reshape(R_EDGE, MAX_NEIGH, HIDDEN).sum(axis=1)
        cnt_ref[...] = v_ref[...].reshape(R_EDGE, MAX_NEIGH).sum(
            axis=1, keepdims=True)

    return pl.pallas_call(
        body,
        grid=(N_ATOMS // R_EDGE,),
        in_specs=[
            pl.BlockSpec((R_EDGE, HIDDEN), lambda g: (g, 0)),
            pl.BlockSpec((re, HIDDEN), lambda g: (g, 0)),
            pl.BlockSpec((1, re), lambda g: (0, g)),
            pl.BlockSpec((re, 1), lambda g: (g, 0)),
            pl.BlockSpec((8, HIDDEN), lambda g: (0, 0)),
            pl.BlockSpec((1, HIDDEN), lambda g: (0, 0)),
            pl.BlockSpec((1, 8), lambda g: (0, 0),
                         memory_space=pltpu.SMEM),
        ],
        out_specs=[
            pl.BlockSpec((R_EDGE, HIDDEN), lambda g: (g, 0)),
            pl.BlockSpec((R_EDGE, 1), lambda g: (g, 0)),
        ],
        out_shape=[
            jax.ShapeDtypeStruct((N_ATOMS, HIDDEN), jnp.float32),
            jax.ShapeDtypeStruct((N_ATOMS, 1), jnp.float32),
        ],
    )(a, bg_pad, dist_row, valid_col, w1c, b1row, freq)


def _update(x, hs, cnt, w2t, b2row, u1at, u1bt, ub1row, u2t, ub2row,
            wab_next=None):
    with_next = wab_next is not None

    def body(x_ref, hs_ref, cnt_ref, w2_ref, b2_ref, u1a_ref, u1b_ref,
             ub1_ref, u2_ref, ub2_ref, *rest):
        if with_next:
            wab_ref, o_ref, ab_ref = rest
        else:
            (o_ref,) = rest
        xb = x_ref[...]
        agg = jnp.dot(hs_ref[...], w2_ref[...],
                      preferred_element_type=jnp.float32)
        agg = agg + cnt_ref[...] * b2_ref[...]
        u = (jnp.dot(xb, u1a_ref[...], preferred_element_type=jnp.float32)
             + jnp.dot(agg, u1b_ref[...], preferred_element_type=jnp.float32)
             + ub1_ref[...])
        u = u * jax.nn.sigmoid(u)
        xn = jnp.dot(u, u2_ref[...],
                     preferred_element_type=jnp.float32) + ub2_ref[...] + xb
        if with_next:
            o_ref[...] = xn
            ab_ref[...] = jnp.dot(xn, wab_ref[...],
                                  preferred_element_type=jnp.float32)
        else:
            o_ref[...] = xn

    h = HIDDEN
    in_specs = [
        pl.BlockSpec((R_MM, h), lambda g: (g, 0)),
        pl.BlockSpec((R_MM, h), lambda g: (g, 0)),
        pl.BlockSpec((R_MM, 1), lambda g: (g, 0)),
        pl.BlockSpec((h, h), lambda g: (0, 0)),
        pl.BlockSpec((1, h), lambda g: (0, 0)),
        pl.BlockSpec((h, h), lambda g: (0, 0)),
        pl.BlockSpec((h, h), lambda g: (0, 0)),
        pl.BlockSpec((1, h), lambda g: (0, 0)),
        pl.BlockSpec((h, h), lambda g: (0, 0)),
        pl.BlockSpec((1, h), lambda g: (0, 0)),
    ]
    inputs = [x, hs, cnt, w2t, b2row, u1at, u1bt, ub1row, u2t, ub2row]
    if with_next:
        in_specs.append(pl.BlockSpec((h, 2 * h), lambda g: (0, 0)))
        inputs.append(wab_next)
        out_specs = [pl.BlockSpec((R_MM, h), lambda g: (g, 0)),
                     pl.BlockSpec((R_MM, 2 * h), lambda g: (g, 0))]
        out_shape = [jax.ShapeDtypeStruct((N_ATOMS, h), jnp.float32),
                     jax.ShapeDtypeStruct((N_ATOMS, 2 * h), jnp.float32)]
    else:
        out_specs = pl.BlockSpec((R_MM, h), lambda g: (g, 0))
        out_shape = jax.ShapeDtypeStruct((N_ATOMS, h), jnp.float32)

    return pl.pallas_call(
        body,
        grid=(N_ATOMS // R_MM,),
        in_specs=in_specs,
        out_specs=out_specs,
        out_shape=out_shape,
    )(*inputs)


def _radius_graph_windowed(pos, batch, wpos, wsq, wbatch, wcol, rbatch):
    def body(p_ref, rb_ref, wp_ref, wsq_ref, wb_ref, wc_ref, j_ref, v_ref):
        g = pl.program_id(0)
        pr = p_ref[...]
        rsq = jnp.sum(pr * pr, axis=1, keepdims=True)
        wp = wp_ref[0]
        dots = jax.lax.dot_general(
            pr, wp, (((1,), (1,)), ((), ())),
            preferred_element_type=jnp.float32)
        d2 = jnp.maximum(rsq + wsq_ref[0] - 2.0 * dots, 0.0)
        ridx = (jax.lax.broadcasted_iota(jnp.int32, (R_GRAPH, 1), 0)
                + g * R_GRAPH)
        wc = wc_ref[0]
        mask = ((rb_ref[...] == wb_ref[0]) & (ridx != wc)
                & (d2 < CUTOFF * CUTOFF))
        m = jnp.where(mask, d2, jnp.inf)
        for k in range(MAX_NEIGH):
            mn = jnp.min(m, axis=1, keepdims=True)
            hit = m == mn
            sel = jnp.min(jnp.where(hit, wc, N_ATOMS), axis=1,
                          keepdims=True)
            ok = mn < jnp.inf
            j_ref[:, k:k + 1] = jnp.where(ok, sel, 0)
            v_ref[:, k:k + 1] = jnp.where(ok, 1.0, 0.0)
            m = jnp.where(hit & (wc == sel), jnp.inf, m)

    nb = N_ATOMS // R_GRAPH
    return pl.pallas_call(
        body,
        grid=(nb,),
        in_specs=[
            pl.BlockSpec((R_GRAPH, 3), lambda g: (g, 0)),
            pl.BlockSpec((R_GRAPH, 1), lambda g: (g, 0)),
            pl.BlockSpec((1, C_WIN, 3), lambda g: (g, 0, 0)),
            pl.BlockSpec((1, 1, C_WIN), lambda g: (g, 0, 0)),
            pl.BlockSpec((1, 1, C_WIN), lambda g: (g, 0, 0)),
            pl.BlockSpec((1, 1, C_WIN), lambda g: (g, 0, 0)),
        ],
        out_specs=[
            pl.BlockSpec((R_GRAPH, MAX_NEIGH), lambda g: (g, 0)),
            pl.BlockSpec((R_GRAPH, MAX_NEIGH), lambda g: (g, 0)),
        ],
        out_shape=[
            jax.ShapeDtypeStruct((N_ATOMS, MAX_NEIGH), jnp.int32),
            jax.ShapeDtypeStruct((N_ATOMS, MAX_NEIGH), jnp.float32),
        ],
    )(pos, rbatch, wpos, wsq, wbatch, wcol)


def _radius_graph(pos, batch):
    n = pos.shape[0]
    sq = jnp.sum(pos * pos, axis=1)
    all_idx = jnp.arange(n)
    cols, valids = [], []
    chunk = 2000
    for start in range(0, n, chunk):
        end = start + chunk
        pc = pos[start:end]
        d2 = sq[start:end, None] + sq[None, :] - 2.0 * (pc @ pos.T)
        d2 = jnp.maximum(d2, 0.0)
        same = batch[start:end, None] == batch[None, :]
        not_self = all_idx[start:end, None] != all_idx[None, :]
        mask = same & not_self & (d2 < CUTOFF * CUTOFF)
        d2m = jnp.where(mask, d2, jnp.inf)
        vals, idxs = jax.lax.top_k(-d2m, MAX_NEIGH)
        cols.append(idxs)
        valids.append(vals > -jnp.inf)
    return jnp.concatenate(cols), jnp.concatenate(valids)


def kernel(atom_features, coordinates, batch_id, rbf_freq,
           i0_W1, i0_b1, i0_W2, i0_b2, i1_W1, i1_b1, i1_W2, i1_b2,
           u0_W1, u0_b1, u0_W2, u0_b2, u1_W1, u1_b1, u1_W2, u1_b2):
    h = HIDDEN
    starts = jnp.searchsorted(batch_id, batch_id, side="left")
    ends = jnp.searchsorted(batch_id, batch_id, side="right")
    maxseg = jnp.max(ends - starts)
    base = jnp.minimum(starts[::R_GRAPH], N_ATOMS - C_WIN)
    wcol = base[:, None] + jnp.arange(C_WIN, dtype=jnp.int32)
    sq_all = jnp.sum(coordinates * coordinates, axis=1)
    wpos = coordinates[wcol]
    wsq = sq_all[wcol][:, None, :]
    wbatch = batch_id[wcol][:, None, :]
    wcol3 = wcol[:, None, :]
    rbatch = batch_id[:, None]

    def _win_path(_):
        return _radius_graph_windowed(coordinates, batch_id, wpos, wsq,
                                      wbatch, wcol3, rbatch)

    def _full_path(_):
        jx, vx = _radius_graph(coordinates, batch_id)
        return jx, vx.astype(jnp.float32)

    j_idx, vf = lax.cond(maxseg <= W_SEG, _win_path, _full_path, None)
    valid = vf > 0.5
    jf = j_idx.reshape(-1)
    diff = coordinates[:, None, :] - coordinates[jf].reshape(
        N_ATOMS, MAX_NEIGH, 3)
    dist = jnp.sqrt(jnp.sum(diff * diff, axis=-1))
    dist = jnp.where(valid, dist, 0.5 * CUTOFF)
    dist_row = dist.reshape(1, E_TOTAL)
    valid_col = vf.reshape(E_TOTAL, 1)
    j_pad = jnp.concatenate(
        [jf, jnp.zeros((E_PAD - E_TOTAL,), dtype=jf.dtype)])
    freq = jnp.zeros((1, 8), jnp.float32).at[0, :NUM_RADIAL].set(rbf_freq)

    inter = [(i0_W1, i0_b1, i0_W2, i0_b2), (i1_W1, i1_b1, i1_W2, i1_b2)]
    upd = [(u0_W1, u0_b1, u0_W2, u0_b2), (u1_W1, u1_b1, u1_W2, u1_b2)]

    def wab_of(W1):
        return jnp.concatenate([W1[:, :h].T, W1[:, h:2 * h].T], axis=1)

    x = atom_features
    ab = _proj_kernel(x, wab_of(inter[0][0]))
    for blk in range(2):
        W1, b1, W2, b2 = inter[blk]
        uW1, ub1, uW2, ub2 = upd[blk]
        w1c = jnp.zeros((8, h), jnp.float32).at[:NUM_RADIAL, :].set(
            W1[:, 2 * h:2 * h + NUM_RADIAL].T)
        a = ab[:, :h]
        b = ab[:, h:]
        bg_pad = _sc_gather(b, j_pad)
        hs, cnt = _edge_combine(a, bg_pad, dist_row, valid_col,
                                w1c, b1[None, :], freq)
        wab_next = wab_of(inter[1][0]) if blk == 0 else None
        res = _update(x, hs, cnt, W2.T, b2[None, :], uW1[:, :h].T,
                      uW1[:, h:].T, ub1[None, :], uW2.T, ub2[None, :],
                      wab_next)
        if blk == 0:
            x, ab = res
        else:
            x = res

    n_valid = jnp.sum(valid)
    return jnp.where(n_valid > 0, x, atom_features)


# speedup vs baseline: 1.9178x; 1.1307x over previous
import functools

import jax
import jax.numpy as jnp
from jax import lax
from jax.experimental import pallas as pl
from jax.experimental.pallas import tpu as pltpu
from jax.experimental.pallas import tpu_sc as plsc

N_ATOMS = 10000
HIDDEN = 256
NUM_RADIAL = 6
CUTOFF = 8.0
MAX_NEIGH = 32
E_TOTAL = N_ATOMS * MAX_NEIGH

SC_WORKERS = 32
SC_CHUNK = 128
SC_CHUNKS_PER_W = 80
E_PAD = SC_WORKERS * SC_CHUNKS_PER_W * SC_CHUNK

R_EDGE = 80
R_MM = 400

R_GRAPH = 200
W_SEG = 384
C_WIN = 1024

_ENV_A = -28.0
_ENV_B = 48.0
_ENV_C = -21.0


def _sc_gather(table, idx_pad):
    mesh = plsc.VectorSubcoreMesh(core_axis_name="c", subcore_axis_name="s")

    @functools.partial(
        pl.kernel,
        mesh=mesh,
        out_type=jax.ShapeDtypeStruct((E_PAD, HIDDEN), jnp.float32),
        scratch_types=[
            pltpu.VMEM((SC_CHUNK,), jnp.int32),
            pltpu.VMEM((SC_CHUNK, HIDDEN), jnp.float32),
            pltpu.SemaphoreType.DMA,
        ],
    )
    def gather_kernel(table_hbm, idx_hbm, out_hbm, idx_v, rows_v, sem):
        wid = lax.axis_index("s") * 2 + lax.axis_index("c")

        def body(t, carry):
            base = (wid * SC_CHUNKS_PER_W + t) * SC_CHUNK
            pltpu.sync_copy(idx_hbm.at[pl.ds(base, SC_CHUNK)], idx_v)
            pltpu.async_copy(table_hbm.at[idx_v], rows_v, sem).wait()
            pltpu.sync_copy(rows_v, out_hbm.at[pl.ds(base, SC_CHUNK)])
            return carry

        lax.fori_loop(0, SC_CHUNKS_PER_W, body, 0)

    return gather_kernel(table, idx_pad)


def _proj_kernel(x, w):
    n, k = x.shape
    m = w.shape[1]

    def body(x_ref, w_ref, o_ref):
        o_ref[...] = jnp.dot(x_ref[...], w_ref[...],
                             preferred_element_type=jnp.float32)

    return pl.pallas_call(
        body,
        grid=(n // R_MM,),
        in_specs=[
            pl.BlockSpec((R_MM, k), lambda g: (g, 0)),
            pl.BlockSpec((k, m), lambda g: (0, 0)),
        ],
        out_specs=pl.BlockSpec((R_MM, m), lambda g: (g, 0)),
        out_shape=jax.ShapeDtypeStruct((n, m), jnp.float32),
    )(x, w)


def _edge_combine(a, bg_pad, dist_row, valid_col, w1c, b1row, freq):
    re = R_EDGE * MAX_NEIGH

    def body(a_ref, bg_ref, d_ref, v_ref, w1c_ref, b1_ref, f_ref,
             hs_ref, cnt_ref):
        x = d_ref[...] * (1.0 / CUTOFF)
        inv = 1.0 / x
        x2 = x * x
        x5 = x2 * x2 * x
        env = (inv + _ENV_A * x5 + _ENV_B * x5 * x + _ENV_C * x5 * x2)
        env = jnp.where(x < 1.0, env, 0.0)
        rows = [env * jnp.sin(f_ref[0, r] * x) for r in range(NUM_RADIAL)]
        rows.append(jnp.zeros((2, re), jnp.float32))
        rbf8 = jnp.concatenate(rows, axis=0)
        rbfp = jax.lax.dot_general(
            rbf8, w1c_ref[...], (((0,), (0,)), ((), ())),
            preferred_element_type=jnp.float32)
        arep = jnp.broadcast_to(
            a_ref[...][:, None, :], (R_EDGE, MAX_NEIGH, HIDDEN)
        ).reshape(re, HIDDEN)
        acc = bg_ref[...] + b1_ref[...] + arep + rbfp
        h = acc * jax.nn.sigmoid(acc)
        h = h * v_ref[...]
        hs_ref[...] = h.reshape(R_EDGE, MAX_NEIGH, HIDDEN).sum(axis=1)
        cnt_ref[...] = v_ref[...].reshape(R_EDGE, MAX_NEIGH).sum(
            axis=1, keepdims=True)

    return pl.pallas_call(
        body,
        grid=(N_ATOMS // R_EDGE,),
        in_specs=[
            pl.BlockSpec((R_EDGE, HIDDEN), lambda g: (g, 0)),
            pl.BlockSpec((re, HIDDEN), lambda g: (g, 0)),
            pl.BlockSpec((1, re), lambda g: (0, g)),
            pl.BlockSpec((re, 1), lambda g: (g, 0)),
            pl.BlockSpec((8, HIDDEN), lambda g: (0, 0)),
            pl.BlockSpec((1, HIDDEN), lambda g: (0, 0)),
            pl.BlockSpec((1, 8), lambda g: (0, 0),
                         memory_space=pltpu.SMEM),
        ],
        out_specs=[
            pl.BlockSpec((R_EDGE, HIDDEN), lambda g: (g, 0)),
            pl.BlockSpec((R_EDGE, 1), lambda g: (g, 0)),
        ],
        out_shape=[
            jax.ShapeDtypeStruct((N_ATOMS, HIDDEN), jnp.float32),
            jax.ShapeDtypeStruct((N_ATOMS, 1), jnp.float32),
        ],
    )(a, bg_pad, dist_row, valid_col, w1c, b1row, freq)


def _update(x, hs, cnt, w2t, b2row, u1at, u1bt, ub1row, u2t, ub2row,
            wab_next=None):
    with_next = wab_next is not None

    def body(x_ref, hs_ref, cnt_ref, w2_ref, b2_ref, u1a_ref, u1b_ref,
             ub1_ref, u2_ref, ub2_ref, *rest):
        if with_next:
            wab_ref, o_ref, ab_ref = rest
        else:
            (o_ref,) = rest
        xb = x_ref[...]
        agg = jnp.dot(hs_ref[...], w2_ref[...],
                      preferred_element_type=jnp.float32)
        agg = agg + cnt_ref[...] * b2_ref[...]
        u = (jnp.dot(xb, u1a_ref[...], preferred_element_type=jnp.float32)
             + jnp.dot(agg, u1b_ref[...], preferred_element_type=jnp.float32)
             + ub1_ref[...])
        u = u * jax.nn.sigmoid(u)
        xn = jnp.dot(u, u2_ref[...],
                     preferred_element_type=jnp.float32) + ub2_ref[...] + xb
        if with_next:
            o_ref[...] = xn
            ab_ref[...] = jnp.dot(xn, wab_ref[...],
                                  preferred_element_type=jnp.float32)
        else:
            o_ref[...] = xn

    h = HIDDEN
    in_specs = [
        pl.BlockSpec((R_MM, h), lambda g: (g, 0)),
        pl.BlockSpec((R_MM, h), lambda g: (g, 0)),
        pl.BlockSpec((R_MM, 1), lambda g: (g, 0)),
        pl.BlockSpec((h, h), lambda g: (0, 0)),
        pl.BlockSpec((1, h), lambda g: (0, 0)),
        pl.BlockSpec((h, h), lambda g: (0, 0)),
        pl.BlockSpec((h, h), lambda g: (0, 0)),
        pl.BlockSpec((1, h), lambda g: (0, 0)),
        pl.BlockSpec((h, h), lambda g: (0, 0)),
        pl.BlockSpec((1, h), lambda g: (0, 0)),
    ]
    inputs = [x, hs, cnt, w2t, b2row, u1at, u1bt, ub1row, u2t, ub2row]
    if with_next:
        in_specs.append(pl.BlockSpec((h, 2 * h), lambda g: (0, 0)))
        inputs.append(wab_next)
        out_specs = [pl.BlockSpec((R_MM, h), lambda g: (g, 0)),
                     pl.BlockSpec((R_MM, 2 * h), lambda g: (g, 0))]
        out_shape = [jax.ShapeDtypeStruct((N_ATOMS, h), jnp.float32),
                     jax.ShapeDtypeStruct((N_ATOMS, 2 * h), jnp.float32)]
    else:
        out_specs = pl.BlockSpec((R_MM, h), lambda g: (g, 0))
        out_shape = jax.ShapeDtypeStruct((N_ATOMS, h), jnp.float32)

    return pl.pallas_call(
        body,
        grid=(N_ATOMS // R_MM,),
        in_specs=in_specs,
        out_specs=out_specs,
        out_shape=out_shape,
    )(*inputs)


def _radius_graph_windowed(pos, batch, wpos, wsq, wbatch, wcol, rbatch):
    def body(p_ref, rb_ref, wp_ref, wsq_ref, wb_ref, wc_ref,
             j_ref, v_ref, d2_ref):
        g = pl.program_id(0)
        pr = p_ref[...]
        rsq = jnp.sum(pr * pr, axis=1, keepdims=True)
        wp = wp_ref[0]
        dots = jax.lax.dot_general(
            pr, wp, (((1,), (1,)), ((), ())),
            preferred_element_type=jnp.float32)
        d2 = jnp.maximum(rsq + wsq_ref[0] - 2.0 * dots, 0.0)
        ridx = (jax.lax.broadcasted_iota(jnp.int32, (R_GRAPH, 1), 0)
                + g * R_GRAPH)
        wc = wc_ref[0]
        mask = ((rb_ref[...] == wb_ref[0]) & (ridx != wc)
                & (d2 < CUTOFF * CUTOFF))
        m = jnp.where(mask, d2, jnp.inf)
        for k in range(MAX_NEIGH):
            mn = jnp.min(m, axis=1, keepdims=True)
            hit = m == mn
            sel = jnp.min(jnp.where(hit, wc, N_ATOMS), axis=1,
                          keepdims=True)
            ok = mn < jnp.inf
            j_ref[:, k:k + 1] = jnp.where(ok, sel, 0)
            v_ref[:, k:k + 1] = jnp.where(ok, 1.0, 0.0)
            d2_ref[:, k:k + 1] = jnp.where(ok, mn, 16.0)
            m = jnp.where(hit & (wc == sel), jnp.inf, m)

    nb = N_ATOMS // R_GRAPH
    return pl.pallas_call(
        body,
        grid=(nb,),
        in_specs=[
            pl.BlockSpec((R_GRAPH, 3), lambda g: (g, 0)),
            pl.BlockSpec((R_GRAPH, 1), lambda g: (g, 0)),
            pl.BlockSpec((1, C_WIN, 3), lambda g: (g, 0, 0)),
            pl.BlockSpec((1, 1, C_WIN), lambda g: (g, 0, 0)),
            pl.BlockSpec((1, 1, C_WIN), lambda g: (g, 0, 0)),
            pl.BlockSpec((1, 1, C_WIN), lambda g: (g, 0, 0)),
        ],
        out_specs=[
            pl.BlockSpec((R_GRAPH, MAX_NEIGH), lambda g: (g, 0)),
            pl.BlockSpec((R_GRAPH, MAX_NEIGH), lambda g: (g, 0)),
            pl.BlockSpec((R_GRAPH, MAX_NEIGH), lambda g: (g, 0)),
        ],
        out_shape=[
            jax.ShapeDtypeStruct((N_ATOMS, MAX_NEIGH), jnp.int32),
            jax.ShapeDtypeStruct((N_ATOMS, MAX_NEIGH), jnp.float32),
            jax.ShapeDtypeStruct((N_ATOMS, MAX_NEIGH), jnp.float32),
        ],
    )(pos, rbatch, wpos, wsq, wbatch, wcol)


def _radius_graph(pos, batch):
    n = pos.shape[0]
    sq = jnp.sum(pos * pos, axis=1)
    all_idx = jnp.arange(n)
    cols, valids, d2s = [], [], []
    chunk = 2000
    for start in range(0, n, chunk):
        end = start + chunk
        pc = pos[start:end]
        d2 = sq[start:end, None] + sq[None, :] - 2.0 * (pc @ pos.T)
        d2 = jnp.maximum(d2, 0.0)
        same = batch[start:end, None] == batch[None, :]
        not_self = all_idx[start:end, None] != all_idx[None, :]
        mask = same & not_self & (d2 < CUTOFF * CUTOFF)
        d2m = jnp.where(mask, d2, jnp.inf)
        vals, idxs = jax.lax.top_k(-d2m, MAX_NEIGH)
        cols.append(idxs)
        valids.append(vals > -jnp.inf)
        d2s.append(vals)
    return (jnp.concatenate(cols), jnp.concatenate(valids),
            jnp.concatenate(d2s))


def kernel(atom_features, coordinates, batch_id, rbf_freq,
           i0_W1, i0_b1, i0_W2, i0_b2, i1_W1, i1_b1, i1_W2, i1_b2,
           u0_W1, u0_b1, u0_W2, u0_b2, u1_W1, u1_b1, u1_W2, u1_b2):
    h = HIDDEN
    pos_idx = jnp.arange(N_ATOMS, dtype=jnp.int32)
    chg = jnp.concatenate([
        jnp.zeros((1,), jnp.int32),
        (batch_id[1:] != batch_id[:-1]).astype(jnp.int32)])
    run_start = jax.lax.cummax(jnp.where(chg == 1, pos_idx, 0))
    maxseg = jnp.max(pos_idx - run_start) + 1
    nb = N_ATOMS // R_GRAPH
    sq_all = jnp.sum(coordinates * coordinates, axis=1)
    sbs = [min(max(b * R_GRAPH - W_SEG, 0), N_ATOMS - C_WIN)
           for b in range(nb)]
    wpos = jnp.stack([lax.slice(coordinates, (s, 0), (s + C_WIN, 3))
                      for s in sbs])
    wsq = jnp.stack([lax.slice(sq_all, (s,), (s + C_WIN,))
                     for s in sbs])[:, None, :]
    wbatch = jnp.stack([lax.slice(batch_id, (s,), (s + C_WIN,))
                        for s in sbs])[:, None, :]
    wcol3 = (jnp.asarray(sbs, jnp.int32)[:, None, None]
             + jnp.arange(C_WIN, dtype=jnp.int32)[None, None, :])
    rbatch = batch_id[:, None]

    def _win_path(_):
        return _radius_graph_windowed(coordinates, batch_id, wpos, wsq,
                                      wbatch, wcol3, rbatch)

    def _full_path(_):
        jx, vx, negd2 = _radius_graph(coordinates, batch_id)
        d2x = jnp.where(vx, jnp.maximum(-negd2, 0.0), 16.0)
        return jx, vx.astype(jnp.float32), d2x

    j_idx, vf, d2sel = lax.cond(maxseg <= W_SEG, _win_path, _full_path,
                                None)
    valid = vf > 0.5
    jf = j_idx.reshape(-1)
    dist = jnp.sqrt(d2sel)
    dist_row = dist.reshape(1, E_TOTAL)
    valid_col = vf.reshape(E_TOTAL, 1)
    j_pad = jnp.concatenate(
        [jf, jnp.zeros((E_PAD - E_TOTAL,), dtype=jf.dtype)])
    freq = jnp.zeros((1, 8), jnp.float32).at[0, :NUM_RADIAL].set(rbf_freq)

    inter = [(i0_W1, i0_b1, i0_W2, i0_b2), (i1_W1, i1_b1, i1_W2, i1_b2)]
    upd = [(u0_W1, u0_b1, u0_W2, u0_b2), (u1_W1, u1_b1, u1_W2, u1_b2)]

    def wab_of(W1):
        return jnp.concatenate([W1[:, :h].T, W1[:, h:2 * h].T], axis=1)

    x = atom_features
    ab = _proj_kernel(x, wab_of(inter[0][0]))
    for blk in range(2):
        W1, b1, W2, b2 = inter[blk]
        uW1, ub1, uW2, ub2 = upd[blk]
        w1c = jnp.zeros((8, h), jnp.float32).at[:NUM_RADIAL, :].set(
            W1[:, 2 * h:2 * h + NUM_RADIAL].T)
        a = ab[:, :h]
        b = ab[:, h:]
        bg_pad = _sc_gather(b, j_pad)
        hs, cnt = _edge_combine(a, bg_pad, dist_row, valid_col,
                                w1c, b1[None, :], freq)
        wab_next = wab_of(inter[1][0]) if blk == 0 else None
        res = _update(x, hs, cnt, W2.T, b2[None, :], uW1[:, :h].T,
                      uW1[:, h:].T, ub1[None, :], uW2.T, ub2[None, :],
                      wab_next)
        if blk == 0:
            x, ab = res
        else:
            x = res

    n_valid = jnp.sum(valid)
    return jnp.where(n_valid > 0, x, atom_features)
